# baseline (speedup 1.0000x reference)
import jax
import jax.numpy as jnp
from jax import lax
from jax.experimental import pallas as pl
from jax.experimental.pallas import tpu as pltpu
from jax.experimental.pallas import tpu_sc as plsc

N = 10000
E = 320000
D = 128
H = 4
C = 32
HC = H * C
HH = 2
FH = HH * C
NEG_SLOPE = 0.2

NC = 2
NS = 16
NW = NC * NS
EPW = E // NS
K = 80
SK = 2000
CP = SK // K
NSUP = EPW // SK
NACC = 5120
NPT = NACC // NS


def _proj_body(x_ref, wl_ref, bl_ref, wr_ref, br_ref, xl_ref, xr_ref):
    xb = x_ref[...]
    xl_ref[...] = (
        jnp.dot(xb, wl_ref[...], preferred_element_type=jnp.float32) + bl_ref[...]
    )
    xr_ref[...] = (
        jnp.dot(xb, wr_ref[...], preferred_element_type=jnp.float32) + br_ref[...]
    )


def _project(x, W_l, b_l, W_r, b_r):
    bn = 1000
    grid = (N // bn,)
    return pl.pallas_call(
        _proj_body,
        grid=grid,
        in_specs=[
            pl.BlockSpec((bn, D), lambda i: (i, 0)),
            pl.BlockSpec((D, HC), lambda i: (0, 0)),
            pl.BlockSpec((1, HC), lambda i: (0, 0)),
            pl.BlockSpec((D, HC), lambda i: (0, 0)),
            pl.BlockSpec((1, HC), lambda i: (0, 0)),
        ],
        out_specs=[
            pl.BlockSpec((bn, HC), lambda i: (i, 0)),
            pl.BlockSpec((bn, HC), lambda i: (i, 0)),
        ],
        out_shape=[
            jax.ShapeDtypeStruct((N, HC), jnp.float32),
            jax.ShapeDtypeStruct((N, HC), jnp.float32),
        ],
    )(x, W_l, b_l.reshape(1, HC), W_r, b_r.reshape(1, HC))


def _sc_edge_body(xl_hbm, xr_hbm, src_hbm, dst_hbm, ea_hbm, att_hbm,
                  zeros_hbm, out_hbm, sout_hbm, s_src, s_dst, s_ea,
                  dstm0, dstm1, rl0, rl1, rr0, rr1, mg0, mg1, attv, tbuf,
                  pbuf, stile, acc, semL0, semL1, semR0, semR1,
                  semS0, semS1):
    cid = lax.axis_index("c")
    sid = lax.axis_index("s")
    wid = cid * NS + sid

    pltpu.sync_copy(zeros_hbm.at[pl.ds(sid * NPT, NPT)],
                    acc.at[pl.ds(sid * NPT, NPT)])
    pltpu.sync_copy(att_hbm.at[pl.ds(cid * FH, FH)], attv)
    plsc.subcore_barrier()

    lanes = lax.iota(jnp.int32, 16)
    col_idx = lanes * 16
    smask = lanes < HH
    zero16 = jnp.zeros((16,), jnp.float32)
    base_w = sid * EPW

    def z_body(z):
        stile[pl.ds(z * 16, 16)] = zero16

    plsc.parallel_loop(0, (N * HH) // 16)(z_body)

    bufs = ((rl0, rr0, semL0, semR0, mg0, dstm0, semS0),
            (rl1, rr1, semL1, semR1, mg1, dstm1, semS1))
    choff = cid * FH
    attr = [attv[pl.ds(q * 16, 16)] for q in range(FH // 16)]

    def issue_gathers(c, b):
        off2 = c * K
        rl_b, rr_b, semL_b, semR_b = bufs[b][:4]
        pltpu.async_copy(xl_hbm.at[s_src.at[pl.ds(off2, K)]], rl_b, semL_b)
        pltpu.async_copy(xr_hbm.at[s_dst.at[pl.ds(off2, K)]], rr_b, semR_b)

    def chunk_work(c, b, prefetch):
        off = c * K
        rl_b, rr_b, semL_b, semR_b, mg, dstm_v, semS_b = bufs[b]
        pltpu.make_async_copy(
            xl_hbm.at[s_src.at[pl.ds(off, K)]], rl_b, semL_b).wait()
        pltpu.make_async_copy(
            xr_hbm.at[s_dst.at[pl.ds(off, K)]], rr_b, semR_b).wait()
        if prefetch:
            issue_gathers(c + 1, 1 - b)
        pltpu.make_async_copy(mg, acc.at[dstm_v], semS_b).wait()

        def remap_body(i):
            d = s_dst[pl.ds(off + i * 16, 16)]
            dstm_v[pl.ds(i * 16, 16)] = d >> 1

        plsc.parallel_loop(0, K // 16)(remap_body)

        for g in range(K // 16):
            def p1_body(e):
                eg = g * 16 + e
                for h in range(HH):
                    a0 = rl_b[eg, pl.ds(choff + h * 32, 16)]
                    a1 = rl_b[eg, pl.ds(choff + h * 32 + 16, 16)]
                    b0 = rr_b[eg, pl.ds(choff + h * 32, 16)]
                    b1 = rr_b[eg, pl.ds(choff + h * 32 + 16, 16)]
                    v0 = a0 + b0
                    v1 = a1 + b1
                    l0 = jnp.maximum(v0, NEG_SLOPE * v0)
                    l1 = jnp.maximum(v1, NEG_SLOPE * v1)
                    t = l0 * attr[2 * h] + l1 * attr[2 * h + 1]
                    tbuf[pl.ds(h * 256 + e * 16, 16)] = t

            plsc.parallel_loop(0, 16, unroll=2)(p1_body)

            wvec = s_ea[pl.ds(off + g * 16, 16)]
            dstvec = s_dst[pl.ds(off + g * 16, 16)]
            for h in range(HH):
                accs = [plsc.load_gather(tbuf, [col_idx + (h * 256 + j)])
                        for j in range(4)]
                for j in range(4, 16):
                    accs[j % 4] = accs[j % 4] + plsc.load_gather(
                        tbuf, [col_idx + (h * 256 + j)])
                s = (accs[0] + accs[1]) + (accs[2] + accs[3])
                pv = wvec * jnp.exp(s)
                pbuf[pl.ds(h * 16, 16)] = pv
                plsc.addupdate_scatter(stile, [dstvec * HH + h], pv)

            def p3_body(e):
                eg = g * 16 + e
                dstb = plsc.load_gather(
                    s_dst, [jnp.full((16,), g * 16, jnp.int32) + (off + e)])
                d1 = (dstb[0] & 1) * FH
                d0 = FH - d1
                for h in range(HH):
                    pvb = plsc.load_gather(
                        pbuf, [jnp.full((16,), h * 16, jnp.int32) + e])
                    a0 = rl_b[eg, pl.ds(choff + h * 32, 16)]
                    a1 = rl_b[eg, pl.ds(choff + h * 32 + 16, 16)]
                    mg[eg, pl.ds(d1 + h * 32, 16)] = pvb * a0
                    mg[eg, pl.ds(d1 + h * 32 + 16, 16)] = pvb * a1
                    mg[eg, pl.ds(d0 + h * 32, 16)] = zero16
                    mg[eg, pl.ds(d0 + h * 32 + 16, 16)] = zero16

            plsc.parallel_loop(0, 16, unroll=2)(p3_body)

        pltpu.async_copy(mg, acc.at[dstm_v], semS_b, add=True)

    def zm_body(z):
        mg0[z // (HC // 16), pl.ds((z % (HC // 16)) * 16, 16)] = zero16
        mg1[z // (HC // 16), pl.ds((z % (HC // 16)) * 16, 16)] = zero16

    plsc.parallel_loop(0, K * HC // 16)(zm_body)
    zidx = jnp.zeros((16,), jnp.int32)
    for i in range(K // 16):
        dstm0[pl.ds(i * 16, 16)] = zidx
        dstm1[pl.ds(i * 16, 16)] = zidx
    pltpu.async_copy(mg0, acc.at[dstm0], semS0, add=True)
    pltpu.async_copy(mg1, acc.at[dstm1], semS1, add=True)

    def super_body(sup, carry):
        soff = base_w + sup * SK
        pltpu.sync_copy(src_hbm.at[pl.ds(soff, SK)], s_src)
        pltpu.sync_copy(dst_hbm.at[pl.ds(soff, SK)], s_dst)
        pltpu.sync_copy(ea_hbm.at[pl.ds(soff, SK)], s_ea)
        issue_gathers(0, 0)

        def pair_body(cc, c0):
            chunk_work(cc * 2, 0, prefetch=True)
            chunk_work(cc * 2 + 1, 1, prefetch=True)
            return c0

        lax.fori_loop(0, (CP - 1) // 2, pair_body, 0)
        chunk_work(CP - 1, 0, prefetch=False)
        return carry

    lax.fori_loop(0, NSUP, super_body, 0)
    pltpu.make_async_copy(mg0, acc.at[dstm0], semS0).wait()
    pltpu.make_async_copy(mg1, acc.at[dstm1], semS1).wait()
    pltpu.sync_copy(stile, sout_hbm.at[wid])
    plsc.subcore_barrier()
    pltpu.sync_copy(acc.at[pl.ds(sid * NPT, NPT)],
                    out_hbm.at[cid, pl.ds(sid * NPT, NPT)])


def _sc_edges(xl, xr, src, dst, edge_attr, attv):
    zeros = jnp.zeros((NACC, HC), jnp.float32)
    mesh = plsc.VectorSubcoreMesh(core_axis_name="c", subcore_axis_name="s",
                                  num_cores=NC)
    f = pl.kernel(
        _sc_edge_body,
        out_type=[
            jax.ShapeDtypeStruct((NC, NACC, HC), jnp.float32),
            jax.ShapeDtypeStruct((NW, N * HH), jnp.float32),
        ],
        mesh=mesh,
        compiler_params=pltpu.CompilerParams(needs_layout_passes=False),
        scratch_types=[
            pltpu.VMEM((SK,), jnp.int32),
            pltpu.VMEM((SK,), jnp.int32),
            pltpu.VMEM((SK,), jnp.float32),
            pltpu.VMEM((K,), jnp.int32),
            pltpu.VMEM((K,), jnp.int32),
            pltpu.VMEM((K, HC), jnp.float32),
            pltpu.VMEM((K, HC), jnp.float32),
            pltpu.VMEM((K, HC), jnp.float32),
            pltpu.VMEM((K, HC), jnp.float32),
            pltpu.VMEM((K, HC), jnp.float32),
            pltpu.VMEM((K, HC), jnp.float32),
            pltpu.VMEM((FH,), jnp.float32),
            pltpu.VMEM((HH * 256,), jnp.float32),
            pltpu.VMEM((HH * 16,), jnp.float32),
            pltpu.VMEM((N * HH,), jnp.float32),
            pltpu.VMEM_SHARED((NACC, HC), jnp.float32),
            pltpu.SemaphoreType.DMA,
            pltpu.SemaphoreType.DMA,
            pltpu.SemaphoreType.DMA,
            pltpu.SemaphoreType.DMA,
            pltpu.SemaphoreType.DMA,
            pltpu.SemaphoreType.DMA,
        ],
    )
    return f(xl, xr, src, dst, edge_attr, attv, zeros)


def _combine_body(alo_ref, ahi_ref, sparts_ref, bias_ref, out_ref):
    acc = jnp.concatenate([alo_ref[...], ahi_ref[...]], axis=1)
    s01 = jnp.sum(sparts_ref[:NS], axis=0)
    s23 = jnp.sum(sparts_ref[NS:], axis=0)
    s4 = jnp.concatenate([s01, s23], axis=1)
    recip = 1.0 / (s4 + 1e-16)
    b_r = lax.broadcasted_iota(jnp.int32, (H, HC), 0)
    b_c = lax.broadcasted_iota(jnp.int32, (H, HC), 1)
    M = jnp.where(b_c // C == b_r, 1.0, 0.0)
    sb = jnp.dot(recip, M, preferred_element_type=jnp.float32)
    out_ref[...] = acc * sb + bias_ref[...]


def _combine(alo, ahi, sparts, bias):
    bn = 1000
    grid = (N // bn,)
    return pl.pallas_call(
        _combine_body,
        grid=grid,
        in_specs=[
            pl.BlockSpec((bn, FH), lambda i: (i, 0)),
            pl.BlockSpec((bn, FH), lambda i: (i, 0)),
            pl.BlockSpec((NW, bn, HH), lambda i: (0, i, 0)),
            pl.BlockSpec((1, HC), lambda i: (0, 0)),
        ],
        out_specs=pl.BlockSpec((bn, HC), lambda i: (i, 0)),
        out_shape=jax.ShapeDtypeStruct((N, HC), jnp.float32),
    )(alo, ahi, sparts, bias.reshape(1, HC))


def kernel(x, edge_index, edge_attr, W_l, b_l, W_r, b_r, att, bias):
    xl, xr = _project(x, W_l, b_l, W_r, b_r)
    src = edge_index[0]
    dst = edge_index[1]
    attv = att.reshape(HC)
    parts, sparts = _sc_edges(xl, xr, src, dst, edge_attr, attv)
    alo = parts[0, : N // 2].reshape(N, FH)
    ahi = parts[1, : N // 2].reshape(N, FH)
    return _combine(alo, ahi, sparts.reshape(NW, N, HH), bias)


# speedup vs baseline: 1.0936x; 1.0936x over previous
import jax
import jax.numpy as jnp
from jax import lax
from jax.experimental import pallas as pl
from jax.experimental.pallas import tpu as pltpu
from jax.experimental.pallas import tpu_sc as plsc

N = 10000
E = 320000
D = 128
H = 4
C = 32
HC = H * C
HH = 2
FH = HH * C
NEG_SLOPE = 0.2

NC = 2
NS = 16
NW = NC * NS
EPW = E // NS
K = 80
SK = 2000
CP = SK // K
NSUP = EPW // SK
NACC = 5120
NPT = NACC // NS


def _proj_body(x_ref, wl_ref, bl_ref, wr_ref, br_ref, xl_ref, xr_ref):
    xb = x_ref[...]
    xl_ref[...] = (
        jnp.dot(xb, wl_ref[...], preferred_element_type=jnp.float32) + bl_ref[...]
    )
    xr_ref[...] = (
        jnp.dot(xb, wr_ref[...], preferred_element_type=jnp.float32) + br_ref[...]
    )


def _project(x, W_l, b_l, W_r, b_r):
    bn = 1000
    grid = (N // bn,)
    return pl.pallas_call(
        _proj_body,
        grid=grid,
        in_specs=[
            pl.BlockSpec((bn, D), lambda i: (i, 0)),
            pl.BlockSpec((D, HC), lambda i: (0, 0)),
            pl.BlockSpec((1, HC), lambda i: (0, 0)),
            pl.BlockSpec((D, HC), lambda i: (0, 0)),
            pl.BlockSpec((1, HC), lambda i: (0, 0)),
        ],
        out_specs=[
            pl.BlockSpec((bn, HC), lambda i: (i, 0)),
            pl.BlockSpec((bn, HC), lambda i: (i, 0)),
        ],
        out_shape=[
            jax.ShapeDtypeStruct((N, HC), jnp.float32),
            jax.ShapeDtypeStruct((N, HC), jnp.float32),
        ],
    )(x, W_l, b_l.reshape(1, HC), W_r, b_r.reshape(1, HC))


def _sc_edge_body(xl_hbm, xr_hbm, src_hbm, dst_hbm, ea_hbm, att_hbm,
                  zeros_hbm, out_hbm, sout_hbm, s_src, s_dst, s_ea,
                  dstm0, dstm1, rl0, rl1, rr0, rr1, mg0, mg1, attv, tbuf,
                  pbuf, stile, acc, semL0, semL1, semR0, semR1,
                  semS0, semS1):
    cid = lax.axis_index("c")
    sid = lax.axis_index("s")
    wid = cid * NS + sid

    pltpu.sync_copy(zeros_hbm.at[pl.ds(sid * NPT, NPT)],
                    acc.at[pl.ds(sid * NPT, NPT)])
    pltpu.sync_copy(att_hbm.at[pl.ds(cid * FH, FH)], attv)
    plsc.subcore_barrier()

    lanes = lax.iota(jnp.int32, 16)
    col_idx = lanes * 16
    smask = lanes < HH
    zero16 = jnp.zeros((16,), jnp.float32)
    base_w = sid * EPW

    def z_body(z):
        stile[pl.ds(z * 16, 16)] = zero16

    plsc.parallel_loop(0, (N * HH) // 16)(z_body)

    bufs = ((rl0, rr0, semL0, semR0, mg0, dstm0, semS0),
            (rl1, rr1, semL1, semR1, mg1, dstm1, semS1))
    choff = cid * FH
    attr = [attv[pl.ds(q * 16, 16)] for q in range(FH // 16)]

    def issue_gathers(c, b):
        off2 = c * K
        rl_b, rr_b, semL_b, semR_b = bufs[b][:4]
        pltpu.async_copy(xl_hbm.at[s_src.at[pl.ds(off2, K)]], rl_b, semL_b)
        pltpu.async_copy(xr_hbm.at[s_dst.at[pl.ds(off2, K)]], rr_b, semR_b)

    def chunk_work(c, b, prefetch):
        off = c * K
        rl_b, rr_b, semL_b, semR_b, mg, dstm_v, semS_b = bufs[b]
        pltpu.make_async_copy(
            xl_hbm.at[s_src.at[pl.ds(off, K)]], rl_b, semL_b).wait()
        pltpu.make_async_copy(
            xr_hbm.at[s_dst.at[pl.ds(off, K)]], rr_b, semR_b).wait()
        if prefetch:
            issue_gathers(c + 1, 1 - b)
        pltpu.make_async_copy(mg, acc.at[dstm_v], semS_b).wait()

        def remap_body(i):
            d = s_dst[pl.ds(off + i * 16, 16)]
            dstm_v[pl.ds(i * 16, 16)] = d >> 1

        plsc.parallel_loop(0, K // 16)(remap_body)

        for g in range(K // 16):
            def p1_body(e):
                eg = g * 16 + e
                for h in range(HH):
                    a0 = rl_b[eg, pl.ds(choff + h * 32, 16)]
                    a1 = rl_b[eg, pl.ds(choff + h * 32 + 16, 16)]
                    b0 = rr_b[eg, pl.ds(choff + h * 32, 16)]
                    b1 = rr_b[eg, pl.ds(choff + h * 32 + 16, 16)]
                    v0 = a0 + b0
                    v1 = a1 + b1
                    l0 = jnp.maximum(v0, NEG_SLOPE * v0)
                    l1 = jnp.maximum(v1, NEG_SLOPE * v1)
                    t = l0 * attr[2 * h] + l1 * attr[2 * h + 1]
                    tbuf[pl.ds(h * 256 + e * 16, 16)] = t

            plsc.parallel_loop(0, 16)(p1_body)

            wvec = s_ea[pl.ds(off + g * 16, 16)]
            dstvec = s_dst[pl.ds(off + g * 16, 16)]
            for h in range(HH):
                accs = [plsc.load_gather(tbuf, [col_idx + (h * 256 + j)])
                        for j in range(4)]
                for j in range(4, 16):
                    accs[j % 4] = accs[j % 4] + plsc.load_gather(
                        tbuf, [col_idx + (h * 256 + j)])
                s = (accs[0] + accs[1]) + (accs[2] + accs[3])
                pv = wvec * jnp.exp(s)
                pbuf[pl.ds(h * 16, 16)] = pv
                plsc.addupdate_scatter(stile, [dstvec * HH + h], pv)

            def p3_body(e):
                eg = g * 16 + e
                dstb = plsc.load_gather(
                    s_dst, [jnp.full((16,), g * 16, jnp.int32) + (off + e)])
                d1 = (dstb[0] & 1) * FH
                d0 = FH - d1
                for h in range(HH):
                    pvb = plsc.load_gather(
                        pbuf, [jnp.full((16,), h * 16, jnp.int32) + e])
                    a0 = rl_b[eg, pl.ds(choff + h * 32, 16)]
                    a1 = rl_b[eg, pl.ds(choff + h * 32 + 16, 16)]
                    mg[eg, pl.ds(d1 + h * 32, 16)] = pvb * a0
                    mg[eg, pl.ds(d1 + h * 32 + 16, 16)] = pvb * a1
                    mg[eg, pl.ds(d0 + h * 32, 16)] = zero16
                    mg[eg, pl.ds(d0 + h * 32 + 16, 16)] = zero16

            plsc.parallel_loop(0, 16)(p3_body)

        pltpu.async_copy(mg, acc.at[dstm_v], semS_b, add=True)

    def zm_body(z):
        mg0[z // (HC // 16), pl.ds((z % (HC // 16)) * 16, 16)] = zero16
        mg1[z // (HC // 16), pl.ds((z % (HC // 16)) * 16, 16)] = zero16

    plsc.parallel_loop(0, K * HC // 16)(zm_body)
    zidx = jnp.zeros((16,), jnp.int32)
    for i in range(K // 16):
        dstm0[pl.ds(i * 16, 16)] = zidx
        dstm1[pl.ds(i * 16, 16)] = zidx
    pltpu.async_copy(mg0, acc.at[dstm0], semS0, add=True)
    pltpu.async_copy(mg1, acc.at[dstm1], semS1, add=True)

    def super_body(sup, carry):
        soff = base_w + sup * SK
        pltpu.sync_copy(src_hbm.at[pl.ds(soff, SK)], s_src)
        pltpu.sync_copy(dst_hbm.at[pl.ds(soff, SK)], s_dst)
        pltpu.sync_copy(ea_hbm.at[pl.ds(soff, SK)], s_ea)
        issue_gathers(0, 0)

        def pair_body(cc, c0):
            chunk_work(cc * 2, 0, prefetch=True)
            chunk_work(cc * 2 + 1, 1, prefetch=True)
            return c0

        lax.fori_loop(0, (CP - 1) // 2, pair_body, 0)
        chunk_work(CP - 1, 0, prefetch=False)
        return carry

    lax.fori_loop(0, NSUP, super_body, 0)
    pltpu.make_async_copy(mg0, acc.at[dstm0], semS0).wait()
    pltpu.make_async_copy(mg1, acc.at[dstm1], semS1).wait()
    pltpu.sync_copy(stile, sout_hbm.at[wid])
    plsc.subcore_barrier()
    pltpu.sync_copy(acc.at[pl.ds(sid * NPT, NPT)],
                    out_hbm.at[cid, pl.ds(sid * NPT, NPT)])


def _sc_edges(xl, xr, src, dst, edge_attr, attv):
    zeros = jnp.zeros((NACC, HC), jnp.float32)
    mesh = plsc.VectorSubcoreMesh(core_axis_name="c", subcore_axis_name="s",
                                  num_cores=NC)
    f = pl.kernel(
        _sc_edge_body,
        out_type=[
            jax.ShapeDtypeStruct((NC, NACC, HC), jnp.float32),
            jax.ShapeDtypeStruct((NW, N * HH), jnp.float32),
        ],
        mesh=mesh,
        compiler_params=pltpu.CompilerParams(needs_layout_passes=False),
        scratch_types=[
            pltpu.VMEM((SK,), jnp.int32),
            pltpu.VMEM((SK,), jnp.int32),
            pltpu.VMEM((SK,), jnp.float32),
            pltpu.VMEM((K,), jnp.int32),
            pltpu.VMEM((K,), jnp.int32),
            pltpu.VMEM((K, HC), jnp.float32),
            pltpu.VMEM((K, HC), jnp.float32),
            pltpu.VMEM((K, HC), jnp.float32),
            pltpu.VMEM((K, HC), jnp.float32),
            pltpu.VMEM((K, HC), jnp.float32),
            pltpu.VMEM((K, HC), jnp.float32),
            pltpu.VMEM((FH,), jnp.float32),
            pltpu.VMEM((HH * 256,), jnp.float32),
            pltpu.VMEM((HH * 16,), jnp.float32),
            pltpu.VMEM((N * HH,), jnp.float32),
            pltpu.VMEM_SHARED((NACC, HC), jnp.float32),
            pltpu.SemaphoreType.DMA,
            pltpu.SemaphoreType.DMA,
            pltpu.SemaphoreType.DMA,
            pltpu.SemaphoreType.DMA,
            pltpu.SemaphoreType.DMA,
            pltpu.SemaphoreType.DMA,
        ],
    )
    return f(xl, xr, src, dst, edge_attr, attv, zeros)


def _combine_body(alo_ref, ahi_ref, sparts_ref, bias_ref, out_ref):
    acc = jnp.concatenate([alo_ref[...], ahi_ref[...]], axis=1)
    s01 = jnp.sum(sparts_ref[:NS], axis=0)
    s23 = jnp.sum(sparts_ref[NS:], axis=0)
    s4 = jnp.concatenate([s01, s23], axis=1)
    recip = 1.0 / (s4 + 1e-16)
    b_r = lax.broadcasted_iota(jnp.int32, (H, HC), 0)
    b_c = lax.broadcasted_iota(jnp.int32, (H, HC), 1)
    M = jnp.where(b_c // C == b_r, 1.0, 0.0)
    sb = jnp.dot(recip, M, preferred_element_type=jnp.float32)
    out_ref[...] = acc * sb + bias_ref[...]


def _combine(alo, ahi, sparts, bias):
    bn = 1000
    grid = (N // bn,)
    return pl.pallas_call(
        _combine_body,
        grid=grid,
        in_specs=[
            pl.BlockSpec((bn, FH), lambda i: (i, 0)),
            pl.BlockSpec((bn, FH), lambda i: (i, 0)),
            pl.BlockSpec((NW, bn, HH), lambda i: (0, i, 0)),
            pl.BlockSpec((1, HC), lambda i: (0, 0)),
        ],
        out_specs=pl.BlockSpec((bn, HC), lambda i: (i, 0)),
        out_shape=jax.ShapeDtypeStruct((N, HC), jnp.float32),
    )(alo, ahi, sparts, bias.reshape(1, HC))


def kernel(x, edge_index, edge_attr, W_l, b_l, W_r, b_r, att, bias):
    xl, xr = _project(x, W_l, b_l, W_r, b_r)
    src = edge_index[0]
    dst = edge_index[1]
    attv = att.reshape(HC)
    parts, sparts = _sc_edges(xl, xr, src, dst, edge_attr, attv)
    alo = parts[0, : N // 2].reshape(N, FH)
    ahi = parts[1, : N // 2].reshape(N, FH)
    return _combine(alo, ahi, sparts.reshape(NW, N, HH), bias)
